# baseline (speedup 1.0000x reference)
import functools

import jax
import jax.numpy as jnp
from jax import lax
from jax.experimental import pallas as pl
from jax.experimental.pallas import tpu as pltpu
from jax.experimental.pallas import tpu_sc as plsc

EMBED_DIM = 1024
CHAR_EMB = 32
CPT = 32
NCH = 513
SLABS = 4


def _make_slab_kernel(n_s, info):
    nw = info.num_cores * info.num_subcores
    per_w = n_s // nw
    c = 40
    n_chunks = per_w // c
    assert per_w % c == 0 and (c * CPT) % 128 == 0 and n_chunks % 2 == 0
    g = (c * CPT) // 128

    mesh = plsc.VectorSubcoreMesh(core_axis_name="c", subcore_axis_name="s")

    @functools.partial(
        pl.kernel,
        mesh=mesh,
        compiler_params=pltpu.CompilerParams(use_tc_tiling_on_sc=False),
        out_type=jax.ShapeDtypeStruct((n_s * CPT, CHAR_EMB), jnp.float32),
        scratch_types=[
            pltpu.VMEM_SHARED((NCH, CHAR_EMB), jnp.float32),
            pltpu.VMEM((c,), jnp.int32),
            pltpu.VMEM((c, CPT), jnp.int32),
            pltpu.VMEM((g, 128), jnp.int32),
            pltpu.VMEM((c * CPT, CHAR_EMB), jnp.float32),
            pltpu.VMEM((c * CPT, CHAR_EMB), jnp.float32),
            pltpu.SemaphoreType.DMA,
            pltpu.SemaphoreType.DMA,
            pltpu.SemaphoreType.DMA,
            pltpu.SemaphoreType.DMA,
        ],
    )
    def k(flat_hbm, tci_hbm, w_hbm, out_hbm,
          w_sh, tok_v, chars_v, idx_v, emb0, emb1, sem1, sem2, so0, so1):
        wid = lax.axis_index("s") * info.num_cores + lax.axis_index("c")
        base = wid * per_w

        @pl.when(lax.axis_index("s") == 0)
        def _load_table():
            pltpu.sync_copy(w_hbm, w_sh)

        plsc.subcore_barrier()

        def fill(i, emb):
            b = base + i * c
            pltpu.sync_copy(flat_hbm.at[pl.ds(b, c)], tok_v)
            pltpu.async_copy(tci_hbm.at[tok_v], chars_v, sem1).wait()

            def flat_body(q, carry):
                row = q >> 1
                off = (q & 1) << 4
                v = chars_v[row, pl.ds(off, 16)]
                idx_v[(q >> 3), pl.ds(((q & 7) << 4), 16)] = v
                return carry

            lax.fori_loop(0, c * 2, flat_body, 0)

            for j in range(g):
                pltpu.async_copy(
                    w_sh.at[idx_v.at[j]], emb.at[pl.ds(j * 128, 128)], sem2
                )
            for j in range(g):
                pltpu.make_async_copy(
                    w_sh.at[idx_v.at[j]], emb.at[pl.ds(j * 128, 128)], sem2
                ).wait()

        def fire_out(i, emb, so):
            b = base + i * c
            pltpu.async_copy(emb, out_hbm.at[pl.ds(b * CPT, c * CPT)], so)

        def wait_out(i, emb, so):
            b = base + i * c
            pltpu.make_async_copy(
                emb, out_hbm.at[pl.ds(b * CPT, c * CPT)], so
            ).wait()

        fill(0, emb0)
        fire_out(0, emb0, so0)
        fill(1, emb1)
        fire_out(1, emb1, so1)

        def pair_body(p, carry):
            i0 = p * 2
            wait_out(i0 - 2, emb0, so0)
            fill(i0, emb0)
            fire_out(i0, emb0, so0)
            wait_out(i0 - 1, emb1, so1)
            fill(i0 + 1, emb1)
            fire_out(i0 + 1, emb1, so1)
            return carry

        lax.fori_loop(1, n_chunks // 2, pair_body, 0)
        wait_out(n_chunks - 2, emb0, so0)
        wait_out(n_chunks - 1, emb1, so1)

    return k


def kernel(input, tokens_chars_idx, char_emb_weight):
    flat = input.reshape(-1)
    n = flat.shape[0]
    n_s = n // SLABS

    info = plsc.get_sparse_core_info()
    k = _make_slab_kernel(n_s, info)

    out = jnp.zeros((n, EMBED_DIM), jnp.float32)
    for s in range(SLABS):
        o = k(
            lax.slice(flat, (s * n_s,), ((s + 1) * n_s,)),
            tokens_chars_idx,
            char_emb_weight,
        )
        out = lax.dynamic_update_slice(out, o.reshape(n_s, EMBED_DIM), (s * n_s, 0))
    return out


# speedup vs baseline: 1.3191x; 1.3191x over previous
import functools

import jax
import jax.numpy as jnp
from jax import lax
from jax.experimental import pallas as pl
from jax.experimental.pallas import tpu as pltpu
from jax.experimental.pallas import tpu_sc as plsc

EMBED_DIM = 1024
CHAR_EMB = 32
CPT = 32
NCH = 513


def _make_gather_kernel(n_s, info):
    nw = info.num_cores * info.num_subcores
    per_w = n_s // nw
    c = 40
    n_chunks = per_w // c
    assert per_w % c == 0 and (c * CPT) % 128 == 0 and n_chunks % 2 == 0
    g = (c * CPT) // 128

    mesh = plsc.VectorSubcoreMesh(core_axis_name="c", subcore_axis_name="s")

    @functools.partial(
        pl.kernel,
        mesh=mesh,
        compiler_params=pltpu.CompilerParams(use_tc_tiling_on_sc=False),
        out_type=jax.ShapeDtypeStruct((n_s * CPT, CHAR_EMB), jnp.float32),
        scratch_types=[
            pltpu.VMEM_SHARED((NCH, CHAR_EMB), jnp.float32),
            pltpu.VMEM((c,), jnp.int32),
            pltpu.VMEM((c, CPT), jnp.int32),
            pltpu.VMEM((g, 128), jnp.int32),
            pltpu.VMEM((c * CPT, CHAR_EMB), jnp.float32),
            pltpu.VMEM((c * CPT, CHAR_EMB), jnp.float32),
            pltpu.SemaphoreType.DMA,
            pltpu.SemaphoreType.DMA,
            pltpu.SemaphoreType.DMA,
            pltpu.SemaphoreType.DMA,
        ],
    )
    def k(flat_hbm, tci_hbm, w_hbm, out_hbm,
          w_sh, tok_v, chars_v, idx_v, emb0, emb1, sem1, sem2, so0, so1):
        wid = lax.axis_index("s") * info.num_cores + lax.axis_index("c")
        base = wid * per_w

        @pl.when(lax.axis_index("s") == 0)
        def _load_table():
            pltpu.sync_copy(w_hbm, w_sh)

        plsc.subcore_barrier()

        def fill(i, emb):
            b = base + i * c
            pltpu.sync_copy(flat_hbm.at[pl.ds(b, c)], tok_v)
            pltpu.async_copy(tci_hbm.at[tok_v], chars_v, sem1).wait()

            def flat_body(q, carry):
                row = q >> 1
                off = (q & 1) << 4
                v = chars_v[row, pl.ds(off, 16)]
                idx_v[(q >> 3), pl.ds(((q & 7) << 4), 16)] = v
                return carry

            lax.fori_loop(0, c * 2, flat_body, 0)

            for j in range(g):
                pltpu.async_copy(
                    w_sh.at[idx_v.at[j]], emb.at[pl.ds(j * 128, 128)], sem2
                )
            for j in range(g):
                pltpu.make_async_copy(
                    w_sh.at[idx_v.at[j]], emb.at[pl.ds(j * 128, 128)], sem2
                ).wait()

        def fire_out(i, emb, so):
            b = base + i * c
            pltpu.async_copy(emb, out_hbm.at[pl.ds(b * CPT, c * CPT)], so)

        def wait_out(i, emb, so):
            b = base + i * c
            pltpu.make_async_copy(
                emb, out_hbm.at[pl.ds(b * CPT, c * CPT)], so
            ).wait()

        fill(0, emb0)
        fire_out(0, emb0, so0)
        fill(1, emb1)
        fire_out(1, emb1, so1)

        def pair_body(p, carry):
            i0 = p * 2
            wait_out(i0 - 2, emb0, so0)
            fill(i0, emb0)
            fire_out(i0, emb0, so0)
            wait_out(i0 - 1, emb1, so1)
            fill(i0 + 1, emb1)
            fire_out(i0 + 1, emb1, so1)
            return carry

        lax.fori_loop(1, n_chunks // 2, pair_body, 0)
        wait_out(n_chunks - 2, emb0, so0)
        wait_out(n_chunks - 1, emb1, so1)

    return k


def kernel(input, tokens_chars_idx, char_emb_weight):
    flat = input.reshape(-1)
    n = flat.shape[0]

    info = plsc.get_sparse_core_info()
    k = _make_gather_kernel(n, info)

    out = k(flat, tokens_chars_idx, char_emb_weight)
    return out.reshape(n, EMBED_DIM)
